# baseline (speedup 1.0000x reference)
import jax
import jax.numpy as jnp
from jax import lax
from jax.experimental import pallas as pl
from jax.experimental.pallas import tpu as pltpu
from jax.experimental.pallas import tpu_sc as plsc

_N = 10000
_E = 320000
_D = 128

_NC = 2
_NS = 16
_NW = _NC * _NS
_DH = _D // _NC
_EPT = _E // _NS
_K = 128
_NB = 160
_EPAD = _NB * _K - _EPT
_R = 4
_NP = 10240
_RPT = _NP // _NS


def _bn_relu(y, g, be):
    mu = jnp.mean(y, axis=0, keepdims=True)
    var = jnp.mean((y - mu) ** 2, axis=0, keepdims=True)
    return jnp.maximum(g * (y - mu) / jnp.sqrt(var + 1e-5) + be, 0.0)


def _matmul_t(x, w):
    return lax.dot_general(x, w, (((1,), (1,)), ((), ())),
                           preferred_element_type=jnp.float32)


def _mlp_body(x_ref, W1, b1, g1, be1, W2, b2, g2, be2, out_ref):
    x = x_ref[...]
    y = _bn_relu(_matmul_t(x, W1[...]) + b1[...], g1[...], be1[...])
    out_ref[...] = _bn_relu(_matmul_t(y, W2[...]) + b2[...], g2[...], be2[...])


def _combine_body(hself_ref, accp_ref, degp_ref,
                  cW1, cb1, cg1, cbe1, cW2, cb2, cg2, cbe2,
                  out_ref):
    agg = jnp.concatenate([accp_ref[0, :_N], accp_ref[1, :_N]], axis=1)
    deg = jnp.sum(degp_ref[:, :_N], axis=0) * 0.5
    aggm = agg / jnp.maximum(deg[:, None], 1.0)
    hself = hself_ref[...]
    W1 = cW1[...]
    y = (_matmul_t(hself, W1[:, :_D]) + _matmul_t(aggm, W1[:, _D:])
         + cb1[...])
    y = _bn_relu(y, cg1[...], cbe1[...])
    y = _bn_relu(_matmul_t(y, cW2[...]) + cb2[...], cg2[...], cbe2[...])
    out_ref[...] = y


def _sc_body(hsrc2, src_lo, src_hi, dst, zacc_hbm, zdeg_hbm,
             acc_out, deg_out, src_v, dst_v, rows_v, deg_v, acc, sem):
    c = lax.axis_index("c")
    s = lax.axis_index("s")
    wid = s * _NC + c

    base = s * _RPT
    pltpu.sync_copy(zacc_hbm, acc.at[pl.ds(base, _RPT)])
    pltpu.sync_copy(zdeg_hbm, deg_v)

    @pl.when(c == 0)
    def _():
        pltpu.sync_copy(src_lo.at[s], src_v)

    @pl.when(c == 1)
    def _():
        pltpu.sync_copy(src_hi.at[s], src_v)

    pltpu.sync_copy(dst.at[s], dst_v)
    plsc.subcore_barrier()

    one16 = jnp.ones((16,), jnp.float32)

    for b in range(_R):
        pltpu.async_copy(hsrc2.at[src_v.at[b]], rows_v[b], sem)

    def batch_group(jj, carry):
        for b in range(_R):
            j = jj * _R + b
            pltpu.make_async_copy(hsrc2.at[src_v.at[j]], rows_v[b], sem).wait()
            pltpu.sync_copy(rows_v[b], acc.at[dst_v.at[j]], add=True)

            @pl.when(j + _R < _NB)
            def _():
                pltpu.async_copy(hsrc2.at[src_v.at[j + _R]], rows_v[b], sem)

            for k in range(_K // 16):
                idx = dst_v[j, pl.ds(k * 16, 16)]
                plsc.addupdate_scatter(deg_v, [idx], one16)
        return carry

    lax.fori_loop(0, _NB // _R, batch_group, 0)
    plsc.subcore_barrier()

    pltpu.sync_copy(acc.at[pl.ds(base, _RPT)],
                    acc_out.at[c, pl.ds(base, _RPT)])
    pltpu.sync_copy(deg_v, deg_out.at[wid])


_SC_AGG_CACHE = []


def _sc_aggregate_fn():
    if not _SC_AGG_CACHE:
        _SC_AGG_CACHE.append(_build_sc_aggregate())
    return _SC_AGG_CACHE[0]


def _build_sc_aggregate():
    return pl.kernel(
        _sc_body,
        out_type=(
            jax.ShapeDtypeStruct((_NC, _NP, _DH), jnp.float32),
            jax.ShapeDtypeStruct((_NW, _NP), jnp.float32),
        ),
        mesh=plsc.VectorSubcoreMesh(core_axis_name="c", subcore_axis_name="s",
                                    num_cores=_NC, num_subcores=_NS),
        compiler_params=pltpu.CompilerParams(needs_layout_passes=False,
                                             use_tc_tiling_on_sc=False),
        scratch_types=[
        pltpu.VMEM((_NB, _K), jnp.int32),
        pltpu.VMEM((_NB, _K), jnp.int32),
        [pltpu.VMEM((_K, _DH), jnp.float32) for _ in range(_R)],
        pltpu.VMEM((_NP,), jnp.float32),
        pltpu.VMEM_SHARED((_NP, _DH), jnp.float32),
        pltpu.SemaphoreType.DMA,
        ],
    )


def kernel(x, edge_index,
           aggr_W1, aggr_b1, aggr_g1, aggr_be1,
           aggr_W2, aggr_b2, aggr_g2, aggr_be2,
           self_W1, self_b1, self_g1, self_be1,
           self_W2, self_b2, self_g2, self_be2,
           comb_W1, comb_b1, comb_g1, comb_be1,
           comb_W2, comb_b2, comb_g2, comb_be2):
    f32 = jnp.float32
    i32 = jnp.int32
    _mlp_call = pl.pallas_call(
        _mlp_body, out_shape=jax.ShapeDtypeStruct((_N, _D), f32))
    hsrc = _mlp_call(x, aggr_W1, aggr_b1, aggr_g1, aggr_be1,
                     aggr_W2, aggr_b2, aggr_g2, aggr_be2)
    hself = _mlp_call(x, self_W1, self_b1, self_g1, self_be1,
                      self_W2, self_b2, self_g2, self_be2)

    ei = edge_index.astype(i32)
    hsrc2 = jnp.concatenate([hsrc[:, :_DH], hsrc[:, _DH:]], axis=0)
    srcr = jnp.concatenate(
        [ei[0].reshape(_NS, _EPT), jnp.zeros((_NS, _EPAD), i32)],
        axis=1).reshape(_NS, _NB, _K)
    dst3 = jnp.concatenate(
        [ei[1].reshape(_NS, _EPT), jnp.full((_NS, _EPAD), _N, i32)],
        axis=1).reshape(_NS, _NB, _K)
    zacc = jnp.zeros((_RPT, _DH), f32)
    zdeg = jnp.zeros((_NP,), f32)
    accp, degp = _sc_aggregate_fn()(hsrc2, srcr, srcr + _N, dst3, zacc, zdeg)

    out = pl.pallas_call(
        _combine_body,
        out_shape=jax.ShapeDtypeStruct((_N, _D), f32),
    )(hself, accp, degp,
      comb_W1, comb_b1, comb_g1, comb_be1,
      comb_W2, comb_b2, comb_g2, comb_be2)
    return out


# speedup vs baseline: 1.0010x; 1.0010x over previous
import jax
import jax.numpy as jnp
from jax import lax
from jax.experimental import pallas as pl
from jax.experimental.pallas import tpu as pltpu
from jax.experimental.pallas import tpu_sc as plsc

_N = 10000
_E = 320000
_D = 128

_NC = 2
_NS = 16
_NW = _NC * _NS
_DH = _D // _NC
_EPT = _E // _NS
_K = 128
_NB = 160
_EPAD = _NB * _K - _EPT
_R = 4
_NP = 10240
_RPT = _NP // _NS


def _bn_relu(y, g, be):
    mu = jnp.mean(y, axis=0, keepdims=True)
    var = jnp.mean((y - mu) ** 2, axis=0, keepdims=True)
    return jnp.maximum(g * (y - mu) / jnp.sqrt(var + 1e-5) + be, 0.0)


def _matmul_t(x, w):
    return lax.dot_general(x, w, (((1,), (1,)), ((), ())),
                           preferred_element_type=jnp.float32)


def _mlp_body(x_ref, W1, b1, g1, be1, W2, b2, g2, be2, out_ref):
    x = x_ref[...]
    y = _bn_relu(_matmul_t(x, W1[...]) + b1[...], g1[...], be1[...])
    out_ref[...] = _bn_relu(_matmul_t(y, W2[...]) + b2[...], g2[...], be2[...])


def _combine_body(hself_ref, accp_ref, degp_ref,
                  cW1, cb1, cg1, cbe1, cW2, cb2, cg2, cbe2,
                  out_ref):
    agg = jnp.concatenate([accp_ref[0, :_N], accp_ref[1, :_N]], axis=1)
    deg = jnp.sum(degp_ref[:, :_N], axis=0) * 0.5
    aggm = agg / jnp.maximum(deg[:, None], 1.0)
    hself = hself_ref[...]
    W1 = cW1[...]
    y = (_matmul_t(hself, W1[:, :_D]) + _matmul_t(aggm, W1[:, _D:])
         + cb1[...])
    y = _bn_relu(y, cg1[...], cbe1[...])
    y = _bn_relu(_matmul_t(y, cW2[...]) + cb2[...], cg2[...], cbe2[...])
    out_ref[...] = y


def _sc_body(hsrc2, src_lo, src_hi, dst, zacc_hbm, zdeg_hbm,
             acc_out, deg_out, src_v, dst_v, rows_v, deg_v, acc, sem):
    c = lax.axis_index("c")
    s = lax.axis_index("s")
    wid = s * _NC + c

    base = s * _RPT
    pltpu.sync_copy(zacc_hbm, acc.at[pl.ds(base, _RPT)])
    pltpu.sync_copy(zdeg_hbm, deg_v)

    @pl.when(c == 0)
    def _():
        pltpu.sync_copy(src_lo.at[s], src_v)

    @pl.when(c == 1)
    def _():
        pltpu.sync_copy(src_hi.at[s], src_v)

    pltpu.sync_copy(dst.at[s], dst_v)
    plsc.subcore_barrier()

    one16 = jnp.ones((16,), jnp.float32)

    for b in range(_R):
        pltpu.async_copy(hsrc2.at[src_v.at[b]], rows_v[b], sem)

    def batch_group(jj, carry):
        for b in range(_R):
            j = jj * _R + b
            pltpu.make_async_copy(hsrc2.at[src_v.at[j]], rows_v[b], sem).wait()
            pltpu.sync_copy(rows_v[b], acc.at[dst_v.at[j]], add=True)

            @pl.when(j + _R < _NB)
            def _():
                pltpu.async_copy(hsrc2.at[src_v.at[j + _R]], rows_v[b], sem)

            for k in range(_K // 16):
                idx = dst_v[j, pl.ds(k * 16, 16)]
                plsc.addupdate_scatter(deg_v, [idx], one16)
        return carry

    lax.fori_loop(0, _NB // _R, batch_group, 0)
    plsc.subcore_barrier()

    pltpu.sync_copy(acc.at[pl.ds(base, _RPT)],
                    acc_out.at[c, pl.ds(base, _RPT)])
    pltpu.sync_copy(deg_v, deg_out.at[wid])


_SC_AGG_CACHE = []


def _sc_aggregate_fn():
    if not _SC_AGG_CACHE:
        _SC_AGG_CACHE.append(_build_sc_aggregate())
    return _SC_AGG_CACHE[0]


def _build_sc_aggregate():
    return pl.kernel(
        _sc_body,
        out_type=(
            jax.ShapeDtypeStruct((_NC, _NP, _DH), jnp.float32),
            jax.ShapeDtypeStruct((_NW, _NP), jnp.float32),
        ),
        mesh=plsc.VectorSubcoreMesh(core_axis_name="c", subcore_axis_name="s",
                                    num_cores=_NC, num_subcores=_NS),
        compiler_params=pltpu.CompilerParams(needs_layout_passes=False,
                                             use_tc_tiling_on_sc=False),
        scratch_types=[
        pltpu.VMEM((_NB, _K), jnp.int32),
        pltpu.VMEM((_NB, _K), jnp.int32),
        [pltpu.VMEM((_K, _DH), jnp.float32) for _ in range(_R)],
        pltpu.VMEM((_NP,), jnp.float32),
        pltpu.VMEM_SHARED((_NP, _DH), jnp.float32),
        pltpu.SemaphoreType.DMA,
        ],
    )


def kernel(x, edge_index,
           aggr_W1, aggr_b1, aggr_g1, aggr_be1,
           aggr_W2, aggr_b2, aggr_g2, aggr_be2,
           self_W1, self_b1, self_g1, self_be1,
           self_W2, self_b2, self_g2, self_be2,
           comb_W1, comb_b1, comb_g1, comb_be1,
           comb_W2, comb_b2, comb_g2, comb_be2):
    f32 = jnp.float32
    i32 = jnp.int32
    _mlp_call = pl.pallas_call(
        _mlp_body, out_shape=jax.ShapeDtypeStruct((_N, _D), f32))
    hsrc = _mlp_call(x, aggr_W1, aggr_b1, aggr_g1, aggr_be1,
                     aggr_W2, aggr_b2, aggr_g2, aggr_be2)
    hself = _mlp_call(x, self_W1, self_b1, self_g1, self_be1,
                      self_W2, self_b2, self_g2, self_be2)

    ei = edge_index.astype(i32)
    hsrc2 = jnp.concatenate([hsrc[:, :_DH], hsrc[:, _DH:]], axis=0)
    srcr = jnp.concatenate(
        [ei[0].reshape(_NS, _EPT), jnp.zeros((_NS, _EPAD), i32)],
        axis=1).reshape(_NS, _NB, _K)
    fake_dst = _N + (jnp.arange(_EPAD, dtype=i32) % (_NP - _N))
    dst3 = jnp.concatenate(
        [ei[1].reshape(_NS, _EPT),
         jnp.broadcast_to(fake_dst, (_NS, _EPAD))],
        axis=1).reshape(_NS, _NB, _K)
    zacc = jnp.zeros((_RPT, _DH), f32)
    zdeg = jnp.zeros((_NP,), f32)
    accp, degp = _sc_aggregate_fn()(hsrc2, srcr, srcr + _N, dst3, zacc, zdeg)

    out = pl.pallas_call(
        _combine_body,
        out_shape=jax.ShapeDtypeStruct((_N, _D), f32),
    )(hself, accp, degp,
      comb_W1, comb_b1, comb_g1, comb_be1,
      comb_W2, comb_b2, comb_g2, comb_be2)
    return out


# speedup vs baseline: 1.7627x; 1.7610x over previous
import jax
import jax.numpy as jnp
from jax import lax
from jax.experimental import pallas as pl
from jax.experimental.pallas import tpu as pltpu
from jax.experimental.pallas import tpu_sc as plsc

_N = 10000
_E = 320000
_D = 128

_NC = 2
_NS = 16
_NW = _NC * _NS
_DH = _D // _NC
_EPT = _E // _NS
_K = 80
_NB = _EPT // _K
_R = 5
_NP = 10240
_RPT = _NP // _NS


def _bn_relu(y, g, be):
    mu = jnp.mean(y, axis=0, keepdims=True)
    var = jnp.mean((y - mu) ** 2, axis=0, keepdims=True)
    return jnp.maximum(g * (y - mu) / jnp.sqrt(var + 1e-5) + be, 0.0)


def _matmul_t(x, w):
    return lax.dot_general(x, w, (((1,), (1,)), ((), ())),
                           preferred_element_type=jnp.float32)


def _mlp_body(x_ref, W1, b1, g1, be1, W2, b2, g2, be2, out_ref):
    x = x_ref[...]
    y = _bn_relu(_matmul_t(x, W1[...]) + b1[...], g1[...], be1[...])
    out_ref[...] = _bn_relu(_matmul_t(y, W2[...]) + b2[...], g2[...], be2[...])


def _combine_body(hself_ref, accp_ref, degp_ref,
                  cW1, cb1, cg1, cbe1, cW2, cb2, cg2, cbe2,
                  out_ref):
    agg = jnp.concatenate([accp_ref[0, :_N], accp_ref[1, :_N]], axis=1)
    deg = jnp.sum(degp_ref[:, :_N], axis=0) * 0.5
    aggm = agg / jnp.maximum(deg[:, None], 1.0)
    hself = hself_ref[...]
    W1 = cW1[...]
    y = (_matmul_t(hself, W1[:, :_D]) + _matmul_t(aggm, W1[:, _D:])
         + cb1[...])
    y = _bn_relu(y, cg1[...], cbe1[...])
    y = _bn_relu(_matmul_t(y, cW2[...]) + cb2[...], cg2[...], cbe2[...])
    out_ref[...] = y


def _sc_body(hsrc2, src_lo, src_hi, dst, zacc_hbm, zdeg_hbm,
             acc_out, deg_out, src_v, dst_v, rows_v, deg_v, acc, sem):
    c = lax.axis_index("c")
    s = lax.axis_index("s")
    wid = s * _NC + c

    base = s * _RPT
    pltpu.sync_copy(zacc_hbm, acc.at[pl.ds(base, _RPT)])
    pltpu.sync_copy(zdeg_hbm, deg_v)

    @pl.when(c == 0)
    def _():
        pltpu.sync_copy(src_lo.at[s], src_v)

    @pl.when(c == 1)
    def _():
        pltpu.sync_copy(src_hi.at[s], src_v)

    pltpu.sync_copy(dst.at[s], dst_v)
    plsc.subcore_barrier()

    one16 = jnp.ones((16,), jnp.float32)

    for b in range(_R):
        pltpu.async_copy(hsrc2.at[src_v.at[b]], rows_v[b], sem)

    def batch_group(jj, carry):
        for b in range(_R):
            j = jj * _R + b
            pltpu.make_async_copy(hsrc2.at[src_v.at[j]], rows_v[b], sem).wait()
            pltpu.sync_copy(rows_v[b], acc.at[dst_v.at[j]], add=True)

            @pl.when(j + _R < _NB)
            def _():
                pltpu.async_copy(hsrc2.at[src_v.at[j + _R]], rows_v[b], sem)

            for k in range(_K // 16):
                idx = dst_v[j, pl.ds(k * 16, 16)]
                plsc.addupdate_scatter(deg_v, [idx], one16)
        return carry

    lax.fori_loop(0, _NB // _R, batch_group, 0)
    plsc.subcore_barrier()

    pltpu.sync_copy(acc.at[pl.ds(base, _RPT)],
                    acc_out.at[c, pl.ds(base, _RPT)])
    pltpu.sync_copy(deg_v, deg_out.at[wid])


_SC_AGG_CACHE = []


def _sc_aggregate_fn():
    if not _SC_AGG_CACHE:
        _SC_AGG_CACHE.append(_build_sc_aggregate())
    return _SC_AGG_CACHE[0]


def _build_sc_aggregate():
    return pl.kernel(
        _sc_body,
        out_type=(
            jax.ShapeDtypeStruct((_NC, _NP, _DH), jnp.float32),
            jax.ShapeDtypeStruct((_NW, _NP), jnp.float32),
        ),
        mesh=plsc.VectorSubcoreMesh(core_axis_name="c", subcore_axis_name="s",
                                    num_cores=_NC, num_subcores=_NS),
        compiler_params=pltpu.CompilerParams(needs_layout_passes=False,
                                             use_tc_tiling_on_sc=False),
        scratch_types=[
        pltpu.VMEM((_NB, _K), jnp.int32),
        pltpu.VMEM((_NB, _K), jnp.int32),
        [pltpu.VMEM((_K, _DH), jnp.float32) for _ in range(_R)],
        pltpu.VMEM((_NP,), jnp.float32),
        pltpu.VMEM_SHARED((_NP, _DH), jnp.float32),
        pltpu.SemaphoreType.DMA,
        ],
    )


def kernel(x, edge_index,
           aggr_W1, aggr_b1, aggr_g1, aggr_be1,
           aggr_W2, aggr_b2, aggr_g2, aggr_be2,
           self_W1, self_b1, self_g1, self_be1,
           self_W2, self_b2, self_g2, self_be2,
           comb_W1, comb_b1, comb_g1, comb_be1,
           comb_W2, comb_b2, comb_g2, comb_be2):
    f32 = jnp.float32
    i32 = jnp.int32
    _mlp_call = pl.pallas_call(
        _mlp_body, out_shape=jax.ShapeDtypeStruct((_N, _D), f32))
    hsrc = _mlp_call(x, aggr_W1, aggr_b1, aggr_g1, aggr_be1,
                     aggr_W2, aggr_b2, aggr_g2, aggr_be2)
    hself = _mlp_call(x, self_W1, self_b1, self_g1, self_be1,
                      self_W2, self_b2, self_g2, self_be2)

    ei = edge_index.astype(i32)
    hsrc2 = jnp.concatenate([hsrc[:, :_DH], hsrc[:, _DH:]], axis=0)
    srcr = ei[0].reshape(_NS, _NB, _K)
    dst3 = ei[1].reshape(_NS, _NB, _K)
    zacc = jnp.zeros((_RPT, _DH), f32)
    zdeg = jnp.zeros((_NP,), f32)
    accp, degp = _sc_aggregate_fn()(hsrc2, srcr, srcr + _N, dst3, zacc, zdeg)

    out = pl.pallas_call(
        _combine_body,
        out_shape=jax.ShapeDtypeStruct((_N, _D), f32),
    )(hself, accp, degp,
      comb_W1, comb_b1, comb_g1, comb_be1,
      comb_W2, comb_b2, comb_g2, comb_be2)
    return out


# speedup vs baseline: 1.9550x; 1.1091x over previous
import jax
import jax.numpy as jnp
from jax import lax
from jax.experimental import pallas as pl
from jax.experimental.pallas import tpu as pltpu
from jax.experimental.pallas import tpu_sc as plsc

_N = 10000
_E = 320000
_D = 128

_NC = 2
_NS = 16
_NW = _NC * _NS
_DH = _D // _NC
_EPT = _E // _NS
_K = 80
_NB = _EPT // _K
_R = 5
_NP = 10240
_RPT = _NP // _NS


def _bn_relu(y, g, be):
    mu = jnp.mean(y, axis=0, keepdims=True)
    var = jnp.mean((y - mu) ** 2, axis=0, keepdims=True)
    return jnp.maximum(g * (y - mu) / jnp.sqrt(var + 1e-5) + be, 0.0)


def _matmul_t(x, w):
    return lax.dot_general(x, w, (((1,), (1,)), ((), ())),
                           preferred_element_type=jnp.float32)


def _mlp_body(x_ref, W1, b1, g1, be1, W2, b2, g2, be2, out_ref):
    x = x_ref[...]
    y = _bn_relu(_matmul_t(x, W1[...]) + b1[...], g1[...], be1[...])
    out_ref[...] = _bn_relu(_matmul_t(y, W2[...]) + b2[...], g2[...], be2[...])


def _combine_body(hself_ref, accp_ref, degp_ref,
                  cW1, cb1, cg1, cbe1, cW2, cb2, cg2, cbe2,
                  out_ref):
    agg = jnp.concatenate([accp_ref[0, :_N], accp_ref[1, :_N]], axis=1)
    deg = jnp.sum(degp_ref[:, :_N], axis=0) * 0.5
    aggm = agg / jnp.maximum(deg[:, None], 1.0)
    hself = hself_ref[...]
    W1 = cW1[...]
    y = (_matmul_t(hself, W1[:, :_D]) + _matmul_t(aggm, W1[:, _D:])
         + cb1[...])
    y = _bn_relu(y, cg1[...], cbe1[...])
    y = _bn_relu(_matmul_t(y, cW2[...]) + cb2[...], cg2[...], cbe2[...])
    out_ref[...] = y


def _sc_body(hsrc2, ei4, zacc_hbm, zdeg_hbm,
             acc_out, deg_out, src_v, dst_v, rows_v, deg_v, acc, sem):
    c = lax.axis_index("c")
    s = lax.axis_index("s")
    wid = s * _NC + c
    table = hsrc2.at[c]

    base = s * _RPT
    pltpu.sync_copy(zacc_hbm, acc.at[pl.ds(base, _RPT)])
    pltpu.sync_copy(zdeg_hbm, deg_v)

    pltpu.sync_copy(ei4.at[0, s], src_v)
    pltpu.sync_copy(ei4.at[1, s], dst_v)
    plsc.subcore_barrier()

    one16 = jnp.ones((16,), jnp.float32)

    for b in range(_R):
        pltpu.async_copy(table.at[src_v.at[b]], rows_v[b], sem)

    def batch_group(jj, carry):
        for b in range(_R):
            j = jj * _R + b
            pltpu.make_async_copy(table.at[src_v.at[j]], rows_v[b], sem).wait()
            pltpu.sync_copy(rows_v[b], acc.at[dst_v.at[j]], add=True)

            @pl.when(j + _R < _NB)
            def _():
                pltpu.async_copy(table.at[src_v.at[j + _R]], rows_v[b], sem)

            for k in range(_K // 16):
                idx = dst_v[j, pl.ds(k * 16, 16)]
                plsc.addupdate_scatter(deg_v, [idx], one16)
        return carry

    lax.fori_loop(0, _NB // _R, batch_group, 0)
    plsc.subcore_barrier()

    pltpu.sync_copy(acc.at[pl.ds(base, _RPT)],
                    acc_out.at[c, pl.ds(base, _RPT)])
    pltpu.sync_copy(deg_v, deg_out.at[wid])


_SC_AGG_CACHE = []


def _sc_aggregate_fn():
    if not _SC_AGG_CACHE:
        _SC_AGG_CACHE.append(_build_sc_aggregate())
    return _SC_AGG_CACHE[0]


def _build_sc_aggregate():
    return pl.kernel(
        _sc_body,
        out_type=(
            jax.ShapeDtypeStruct((_NC, _NP, _DH), jnp.float32),
            jax.ShapeDtypeStruct((_NW, _NP), jnp.float32),
        ),
        mesh=plsc.VectorSubcoreMesh(core_axis_name="c", subcore_axis_name="s",
                                    num_cores=_NC, num_subcores=_NS),
        compiler_params=pltpu.CompilerParams(needs_layout_passes=False,
                                             use_tc_tiling_on_sc=False),
        scratch_types=[
        pltpu.VMEM((_NB, _K), jnp.int32),
        pltpu.VMEM((_NB, _K), jnp.int32),
        [pltpu.VMEM((_K, _DH), jnp.float32) for _ in range(_R)],
        pltpu.VMEM((_NP,), jnp.float32),
        pltpu.VMEM_SHARED((_NP, _DH), jnp.float32),
        pltpu.SemaphoreType.DMA,
        ],
    )


def kernel(x, edge_index,
           aggr_W1, aggr_b1, aggr_g1, aggr_be1,
           aggr_W2, aggr_b2, aggr_g2, aggr_be2,
           self_W1, self_b1, self_g1, self_be1,
           self_W2, self_b2, self_g2, self_be2,
           comb_W1, comb_b1, comb_g1, comb_be1,
           comb_W2, comb_b2, comb_g2, comb_be2):
    f32 = jnp.float32
    i32 = jnp.int32
    _mlp_call = pl.pallas_call(
        _mlp_body, out_shape=jax.ShapeDtypeStruct((_N, _D), f32))
    hsrc = _mlp_call(x, aggr_W1, aggr_b1, aggr_g1, aggr_be1,
                     aggr_W2, aggr_b2, aggr_g2, aggr_be2)
    hself = _mlp_call(x, self_W1, self_b1, self_g1, self_be1,
                      self_W2, self_b2, self_g2, self_be2)

    hsrc2 = jnp.stack([hsrc[:, :_DH], hsrc[:, _DH:]])
    ei4 = edge_index.astype(i32).reshape(2, _NS, _NB, _K)
    zacc = jnp.zeros((_RPT, _DH), f32)
    zdeg = jnp.zeros((_NP,), f32)
    accp, degp = _sc_aggregate_fn()(hsrc2, ei4, zacc, zdeg)

    out = pl.pallas_call(
        _combine_body,
        out_shape=jax.ShapeDtypeStruct((_N, _D), f32),
    )(hself, accp, degp,
      comb_W1, comb_b1, comb_g1, comb_be1,
      comb_W2, comb_b2, comb_g2, comb_be2)
    return out


# speedup vs baseline: 2.0340x; 1.0404x over previous
import jax
import jax.numpy as jnp
from jax import lax
from jax.experimental import pallas as pl
from jax.experimental.pallas import tpu as pltpu
from jax.experimental.pallas import tpu_sc as plsc

_N = 10000
_E = 320000
_D = 128

_NC = 2
_NS = 16
_NW = _NC * _NS
_DH = _D // _NC
_EPT = _E // _NS
_K = 80
_NB = _EPT // _K
_R = 5
_NP = 10240
_RPT = _NP // _NS


def _bn_relu(y, g, be):
    mu = jnp.mean(y, axis=0, keepdims=True)
    var = jnp.mean((y - mu) ** 2, axis=0, keepdims=True)
    return jnp.maximum(g * (y - mu) / jnp.sqrt(var + 1e-5) + be, 0.0)


def _matmul_t(x, w):
    return lax.dot_general(x, w, (((1,), (1,)), ((), ())),
                           preferred_element_type=jnp.float32)


def _mlp_body(x_ref, W1, b1, g1, be1, W2, b2, g2, be2, out_ref):
    x = x_ref[...]
    y = _bn_relu(_matmul_t(x, W1[...]) + b1[...], g1[...], be1[...])
    out_ref[...] = _bn_relu(_matmul_t(y, W2[...]) + b2[...], g2[...], be2[...])


def _mlp_split_body(x_ref, W1, b1, g1, be1, W2, b2, g2, be2, out_ref):
    x = x_ref[...]
    y = _bn_relu(_matmul_t(x, W1[...]) + b1[...], g1[...], be1[...])
    h = _bn_relu(_matmul_t(y, W2[...]) + b2[...], g2[...], be2[...])
    out_ref[0] = h[:, :_DH]
    out_ref[1] = h[:, _DH:]


def _combine_body(hself_ref, accp_ref, degp_ref,
                  cW1, cb1, cg1, cbe1, cW2, cb2, cg2, cbe2,
                  out_ref):
    agg = jnp.concatenate([accp_ref[0, :_N], accp_ref[1, :_N]], axis=1)
    deg = jnp.sum(degp_ref[:, :_N], axis=0) * 0.5
    aggm = agg / jnp.maximum(deg[:, None], 1.0)
    hself = hself_ref[...]
    W1 = cW1[...]
    y = (_matmul_t(hself, W1[:, :_D]) + _matmul_t(aggm, W1[:, _D:])
         + cb1[...])
    y = _bn_relu(y, cg1[...], cbe1[...])
    y = _bn_relu(_matmul_t(y, cW2[...]) + cb2[...], cg2[...], cbe2[...])
    out_ref[...] = y


def _sc_body(hsrc2, ei4, zacc_hbm, zdeg_hbm,
             acc_out, deg_out, src_v, dst_v, rows_v, deg_v, acc, sem):
    c = lax.axis_index("c")
    s = lax.axis_index("s")
    wid = s * _NC + c
    table = hsrc2.at[c]

    base = s * _RPT
    pltpu.sync_copy(zacc_hbm, acc.at[pl.ds(base, _RPT)])
    pltpu.sync_copy(zdeg_hbm, deg_v)

    pltpu.sync_copy(ei4.at[0, s], src_v)
    pltpu.sync_copy(ei4.at[1, s], dst_v)
    plsc.subcore_barrier()

    one16 = jnp.ones((16,), jnp.float32)

    for b in range(_R):
        pltpu.async_copy(table.at[src_v.at[b]], rows_v[b], sem)

    def batch_group(jj, carry):
        for b in range(_R):
            j = jj * _R + b
            pltpu.make_async_copy(table.at[src_v.at[j]], rows_v[b], sem).wait()
            pltpu.sync_copy(rows_v[b], acc.at[dst_v.at[j]], add=True)

            @pl.when(j + _R < _NB)
            def _():
                pltpu.async_copy(table.at[src_v.at[j + _R]], rows_v[b], sem)

            for k in range(_K // 16):
                idx = dst_v[j, pl.ds(k * 16, 16)]
                plsc.addupdate_scatter(deg_v, [idx], one16)
        return carry

    lax.fori_loop(0, _NB // _R, batch_group, 0)
    plsc.subcore_barrier()

    pltpu.sync_copy(acc.at[pl.ds(base, _RPT)],
                    acc_out.at[c, pl.ds(base, _RPT)])
    pltpu.sync_copy(deg_v, deg_out.at[wid])


_SC_AGG_CACHE = []


def _sc_aggregate_fn():
    if not _SC_AGG_CACHE:
        _SC_AGG_CACHE.append(_build_sc_aggregate())
    return _SC_AGG_CACHE[0]


def _build_sc_aggregate():
    return pl.kernel(
        _sc_body,
        out_type=(
            jax.ShapeDtypeStruct((_NC, _NP, _DH), jnp.float32),
            jax.ShapeDtypeStruct((_NW, _NP), jnp.float32),
        ),
        mesh=plsc.VectorSubcoreMesh(core_axis_name="c", subcore_axis_name="s",
                                    num_cores=_NC, num_subcores=_NS),
        compiler_params=pltpu.CompilerParams(needs_layout_passes=False,
                                             use_tc_tiling_on_sc=False),
        scratch_types=[
        pltpu.VMEM((_NB, _K), jnp.int32),
        pltpu.VMEM((_NB, _K), jnp.int32),
        [pltpu.VMEM((_K, _DH), jnp.float32) for _ in range(_R)],
        pltpu.VMEM((_NP,), jnp.float32),
        pltpu.VMEM_SHARED((_NP, _DH), jnp.float32),
        pltpu.SemaphoreType.DMA,
        ],
    )


def kernel(x, edge_index,
           aggr_W1, aggr_b1, aggr_g1, aggr_be1,
           aggr_W2, aggr_b2, aggr_g2, aggr_be2,
           self_W1, self_b1, self_g1, self_be1,
           self_W2, self_b2, self_g2, self_be2,
           comb_W1, comb_b1, comb_g1, comb_be1,
           comb_W2, comb_b2, comb_g2, comb_be2):
    f32 = jnp.float32
    i32 = jnp.int32
    hsrc2 = pl.pallas_call(
        _mlp_split_body,
        out_shape=jax.ShapeDtypeStruct((_NC, _N, _DH), f32),
    )(x, aggr_W1, aggr_b1, aggr_g1, aggr_be1,
      aggr_W2, aggr_b2, aggr_g2, aggr_be2)
    hself = pl.pallas_call(
        _mlp_body, out_shape=jax.ShapeDtypeStruct((_N, _D), f32),
    )(x, self_W1, self_b1, self_g1, self_be1,
      self_W2, self_b2, self_g2, self_be2)
    ei4 = edge_index.astype(i32).reshape(2, _NS, _NB, _K)
    zacc = jnp.zeros((_RPT, _DH), f32)
    zdeg = jnp.zeros((_NP,), f32)
    accp, degp = _sc_aggregate_fn()(hsrc2, ei4, zacc, zdeg)

    out = pl.pallas_call(
        _combine_body,
        out_shape=jax.ShapeDtypeStruct((_N, _D), f32),
    )(hself, accp, degp,
      comb_W1, comb_b1, comb_g1, comb_be1,
      comb_W2, comb_b2, comb_g2, comb_be2)
    return out
